# Initial kernel scaffold; baseline (speedup 1.0000x reference)
import jax
import jax.numpy as jnp
from jax.experimental import pallas as pl


def kernel(x_noisy, t, atom_types, edge_index, bond_types, batch_idx, params):
    raise NotImplementedError("write your pallas kernel here")


# speedup vs baseline: 1.5791x; 1.5791x over previous
import functools

import jax
import jax.numpy as jnp
import numpy as np
from jax import lax
from jax.experimental import pallas as pl
from jax.experimental.pallas import tpu as pltpu
from jax.experimental.pallas import tpu_sc as plsc

F32 = jnp.float32
I32 = jnp.int32

HID = 128
TD = 128
NL = 6
N = 10000
E = 320000
B = 64

NP = 10240
EP = 327680
WIN = 128
NWORK = 32
EPW = EP // NWORK
NWIN = EPW // WIN
ROWS_PER_TILE = NP // 16

BE = 4096
BN = 1024


def _silu(x):
    return x * (1.0 / (1.0 + jnp.exp(-x)))


def _dot(a, b):
    return jnp.dot(a, b, preferred_element_type=F32)


@functools.cache
def _sc_mesh():
    return plsc.VectorSubcoreMesh(core_axis_name="c", subcore_axis_name="s")


@jax.jit
def _sc_gather(trow, tcol, xtab, row, col):
    @functools.partial(
        pl.kernel,
        out_type=(
            jax.ShapeDtypeStruct((EP, HID), F32),
            jax.ShapeDtypeStruct((EP, HID), F32),
            jax.ShapeDtypeStruct((EP, HID), F32),
            jax.ShapeDtypeStruct((EP, HID), F32),
        ),
        mesh=_sc_mesh(),
        scratch_types=[
            pltpu.VMEM((WIN,), I32),
            pltpu.VMEM((WIN,), I32),
            pltpu.VMEM((WIN, HID), F32),
            pltpu.VMEM((WIN, HID), F32),
            pltpu.VMEM((WIN, HID), F32),
            pltpu.VMEM((WIN, HID), F32),
            pltpu.SemaphoreType.DMA,
        ],
    )
    def k(trow_hbm, tcol_hbm, xtab_hbm, row_hbm, col_hbm,
          g1_hbm, g2_hbm, gx1_hbm, gx2_hbm,
          idx1, idx2, buf1, buf2, bufx1, bufx2, sem):
        wid = lax.axis_index("s") * 2 + lax.axis_index("c")
        base = wid * EPW

        @pl.loop(0, NWIN)
        def _(w):
            off = base + w * WIN
            pltpu.sync_copy(row_hbm.at[pl.ds(off, WIN)], idx1)
            pltpu.sync_copy(col_hbm.at[pl.ds(off, WIN)], idx2)
            cp1 = pltpu.async_copy(trow_hbm.at[idx1], buf1, sem)
            cp2 = pltpu.async_copy(tcol_hbm.at[idx2], buf2, sem)
            cp3 = pltpu.async_copy(xtab_hbm.at[idx1], bufx1, sem)
            cp4 = pltpu.async_copy(xtab_hbm.at[idx2], bufx2, sem)
            cp1.wait()
            cp2.wait()
            cp3.wait()
            cp4.wait()
            pltpu.sync_copy(buf1, g1_hbm.at[pl.ds(off, WIN)])
            pltpu.sync_copy(buf2, g2_hbm.at[pl.ds(off, WIN)])
            pltpu.sync_copy(bufx1, gx1_hbm.at[pl.ds(off, WIN)])
            pltpu.sync_copy(bufx2, gx2_hbm.at[pl.ds(off, WIN)])

    return k(trow, tcol, xtab, row, col)


@jax.jit
def _sc_scatter(payload, col, zeros_tbl):
    @functools.partial(
        pl.kernel,
        out_type=jax.ShapeDtypeStruct((2, NP, HID), F32),
        mesh=_sc_mesh(),
        scratch_types=[
            pltpu.VMEM((WIN,), I32),
            pltpu.VMEM((WIN, HID), F32),
            pltpu.VMEM_SHARED((NP, HID), F32),
            pltpu.SemaphoreType.DMA,
        ],
    )
    def k(p_hbm, col_hbm, z_hbm, out_hbm, idxb, rowb, acc, sem):
        c = lax.axis_index("c")
        s = lax.axis_index("s")
        rbase = s * ROWS_PER_TILE
        pltpu.sync_copy(z_hbm.at[pl.ds(rbase, ROWS_PER_TILE)],
                        acc.at[pl.ds(rbase, ROWS_PER_TILE)])
        plsc.subcore_barrier()
        base = c * (EP // 2) + s * EPW

        @pl.loop(0, NWIN)
        def _(w):
            off = base + w * WIN
            pltpu.sync_copy(col_hbm.at[pl.ds(off, WIN)], idxb)
            pltpu.async_copy(p_hbm.at[pl.ds(off, WIN)], rowb, sem).wait()
            pltpu.sync_copy(rowb, acc.at[idxb], add=True)

        plsc.subcore_barrier()
        pltpu.sync_copy(acc.at[pl.ds(rbase, ROWS_PER_TILE)],
                        out_hbm.at[c, pl.ds(rbase, ROWS_PER_TILE)])

    return k(payload, col, zeros_tbl)


def _te_body(t_ref, w1_ref, b1_ref, w2_ref, b2_ref, te_ref):
    tt = t_ref[...]
    half = TD // 2
    k = lax.broadcasted_iota(I32, (B, half), 1).astype(F32)
    freq = jnp.exp(k * (-np.log(10000.0) / (half - 1)))
    emb = tt * freq
    e1 = jnp.concatenate([jnp.sin(emb), jnp.cos(emb)], axis=1)
    te = _dot(_silu(_dot(e1, w1_ref[...]) + b1_ref[...]), w2_ref[...]) + b2_ref[...]
    te_ref[...] = te


@jax.jit
def _tc_te(t2, w1, b1, w2, b2):
    return pl.pallas_call(
        _te_body,
        out_shape=jax.ShapeDtypeStruct((B, HID), F32),
    )(t2, w1, b1, w2, b2)


def _prologue_body(x_ref, at_ref, bi_ref, te_ref, ae_ref, cw_ref, cb_ref,
                   wr_ref, wc_ref, h_ref, trow_ref, tcol_ref, xtab_ref):
    x = x_ref[...]
    at = jnp.clip(at_ref[...], 0, 9)
    oha = (at == lax.broadcasted_iota(I32, (BN, 16), 1)).astype(F32)
    ohb = (bi_ref[...] == lax.broadcasted_iota(I32, (BN, B), 1)).astype(F32)
    h = _dot(oha, ae_ref[...]) + _dot(x, cw_ref[...]) + cb_ref[...] \
        + _dot(ohb, te_ref[...])
    h_ref[...] = h
    trow_ref[...] = _dot(h, wr_ref[...])
    tcol_ref[...] = _dot(h, wc_ref[...])
    xtab_ref[...] = jnp.pad(x, ((0, 0), (0, HID - 16)))


@jax.jit
def _tc_prologue(x16, at, bi, te, ae16, cw16, cb, wr, wc):
    nb = NP // BN
    full = lambda shape: pl.BlockSpec(shape, lambda i: (0,) * len(shape))
    return pl.pallas_call(
        _prologue_body,
        grid=(nb,),
        in_specs=[
            pl.BlockSpec((BN, 16), lambda i: (i, 0)),
            pl.BlockSpec((BN, 1), lambda i: (i, 0)),
            pl.BlockSpec((BN, 1), lambda i: (i, 0)),
            full((B, HID)),
            full((16, HID)),
            full((16, HID)),
            full((1, HID)),
            full((HID, HID)),
            full((HID, HID)),
        ],
        out_specs=[
            pl.BlockSpec((BN, HID), lambda i: (i, 0)),
            pl.BlockSpec((BN, HID), lambda i: (i, 0)),
            pl.BlockSpec((BN, HID), lambda i: (i, 0)),
            pl.BlockSpec((BN, HID), lambda i: (i, 0)),
        ],
        out_shape=[
            jax.ShapeDtypeStruct((NP, HID), F32),
            jax.ShapeDtypeStruct((NP, HID), F32),
            jax.ShapeDtypeStruct((NP, HID), F32),
            jax.ShapeDtypeStruct((NP, HID), F32),
        ],
    )(x16, at, bi, te, ae16, cw16, cb, wr, wc)


def _edge_body(g1_ref, g2_ref, gx1_ref, gx2_ref, bt_ref, bond8_ref, wea_ref,
               wd_ref, be1_ref, we2_ref, be2_ref, wc1_ref, bc1_ref, wc2_ref,
               bc2_ref, p_ref, cwu_ref):
    hsum = g1_ref[...] + g2_ref[...]
    dx3 = gx1_ref[:, :3] - gx2_ref[:, :3]
    d2 = jnp.sum(dx3 * dx3, axis=-1, keepdims=True)
    dist = jnp.maximum(jnp.sqrt(d2), 1e-6)
    unit = dx3 / dist
    btc = jnp.clip(bt_ref[...], 0, 4)
    oh = (btc == lax.broadcasted_iota(I32, (BE, 8), 1)).astype(F32)
    tbl8 = _dot(bond8_ref[...], wea_ref[...])
    pre = hsum + _dot(oh, tbl8) + dist * wd_ref[...] + be1_ref[...]
    t1 = _silu(pre)
    m = _silu(_dot(t1, we2_ref[...]) + be2_ref[...])
    c1 = _silu(_dot(m, wc1_ref[...]) + bc1_ref[...])
    cw = jnp.sum(c1 * wc2_ref[...], axis=-1, keepdims=True) + bc2_ref[...]
    p_ref[...] = m
    cwu_ref[...] = jnp.pad(cw * unit, ((0, 0), (0, HID - 3)))


@jax.jit
def _tc_edge(g1, g2, gx1, gx2, bt, bond8, wea, wd, be1, we2, be2, wc1, bc1,
             wc2r, bc2):
    nb = EP // BE
    full = lambda shape: pl.BlockSpec(shape, lambda i: (0,) * len(shape))
    return pl.pallas_call(
        _edge_body,
        grid=(nb,),
        in_specs=[
            pl.BlockSpec((BE, HID), lambda i: (i, 0)),
            pl.BlockSpec((BE, HID), lambda i: (i, 0)),
            pl.BlockSpec((BE, HID), lambda i: (i, 0)),
            pl.BlockSpec((BE, HID), lambda i: (i, 0)),
            pl.BlockSpec((BE, 1), lambda i: (i, 0)),
            full((8, 32)),
            full((32, HID)),
            full((1, HID)),
            full((1, HID)),
            full((HID, HID)),
            full((1, HID)),
            full((HID, 64)),
            full((1, 64)),
            full((1, 64)),
            full((1, 1)),
        ],
        out_specs=[
            pl.BlockSpec((BE, HID), lambda i: (i, 0)),
            pl.BlockSpec((BE, HID), lambda i: (i, 0)),
        ],
        out_shape=[
            jax.ShapeDtypeStruct((EP, HID), F32),
            jax.ShapeDtypeStruct((EP, HID), F32),
        ],
    )(g1, g2, gx1, gx2, bt, bond8, wea, wd, be1, we2, be2, wc1, bc1, wc2r,
      bc2)


def _node_body(h_ref, x_ref, pm_ref, px_ref, wn1h_ref, wn1m_ref, bn1_ref,
               wn2_ref, bn2_ref, lng_ref, lnb_ref, wr_ref, wc_ref,
               ho_ref, trow_ref, tcol_ref, xtab_ref):
    h = h_ref[...]
    pm = pm_ref[...]
    magg = pm[0] + pm[1]
    xd = px_ref[0, :, :3] + px_ref[1, :, :3]
    xn = x_ref[:, :3] + xd
    n1 = _silu(_dot(h, wn1h_ref[...]) + _dot(magg, wn1m_ref[...]) + bn1_ref[...])
    hn = _dot(n1, wn2_ref[...]) + bn2_ref[...]
    s = h + hn
    mu = jnp.mean(s, axis=-1, keepdims=True)
    var = jnp.mean((s - mu) ** 2, axis=-1, keepdims=True)
    ho = (s - mu) / jnp.sqrt(var + 1e-5) * lng_ref[...] + lnb_ref[...]
    ho_ref[...] = ho
    trow_ref[...] = _dot(ho, wr_ref[...])
    tcol_ref[...] = _dot(ho, wc_ref[...])
    xtab_ref[...] = jnp.pad(xn, ((0, 0), (0, HID - 3)))


@jax.jit
def _tc_node(h, xtab, pm, px, wn1h, wn1m, bn1, wn2, bn2, lng, lnb, wr, wc):
    nb = NP // BN
    full = lambda shape: pl.BlockSpec(shape, lambda i: (0,) * len(shape))
    return pl.pallas_call(
        _node_body,
        grid=(nb,),
        in_specs=[
            pl.BlockSpec((BN, HID), lambda i: (i, 0)),
            pl.BlockSpec((BN, HID), lambda i: (i, 0)),
            pl.BlockSpec((2, BN, HID), lambda i: (0, i, 0)),
            pl.BlockSpec((2, BN, HID), lambda i: (0, i, 0)),
            full((HID, HID)),
            full((HID, HID)),
            full((1, HID)),
            full((HID, HID)),
            full((1, HID)),
            full((1, HID)),
            full((1, HID)),
            full((HID, HID)),
            full((HID, HID)),
        ],
        out_specs=[
            pl.BlockSpec((BN, HID), lambda i: (i, 0)),
            pl.BlockSpec((BN, HID), lambda i: (i, 0)),
            pl.BlockSpec((BN, HID), lambda i: (i, 0)),
            pl.BlockSpec((BN, HID), lambda i: (i, 0)),
        ],
        out_shape=[
            jax.ShapeDtypeStruct((NP, HID), F32),
            jax.ShapeDtypeStruct((NP, HID), F32),
            jax.ShapeDtypeStruct((NP, HID), F32),
            jax.ShapeDtypeStruct((NP, HID), F32),
        ],
    )(h, xtab, pm, px, wn1h, wn1m, bn1, wn2, bn2, lng, lnb, wr, wc)


def _pred_body(h_ref, w1_ref, b1_ref, w2_ref, b2_ref, o_ref):
    t1 = _silu(_dot(h_ref[...], w1_ref[...]) + b1_ref[...])
    o_ref[...] = _dot(t1, w2_ref[...]) + b2_ref[...]


@jax.jit
def _tc_pred(h, w1, b1, w2p, b2p):
    nb = NP // BN
    full = lambda shape: pl.BlockSpec(shape, lambda i: (0,) * len(shape))
    return pl.pallas_call(
        _pred_body,
        grid=(nb,),
        in_specs=[
            pl.BlockSpec((BN, HID), lambda i: (i, 0)),
            full((HID, HID)),
            full((1, HID)),
            full((HID, HID)),
            full((1, HID)),
        ],
        out_specs=pl.BlockSpec((BN, HID), lambda i: (i, 0)),
        out_shape=jax.ShapeDtypeStruct((NP, HID), F32),
    )(h, w1, b1, w2p, b2p)


def kernel(x_noisy, t, atom_types, edge_index, bond_types, batch_idx, params):
    p = params

    x16 = jnp.pad(x_noisy.astype(F32), ((0, NP - N), (0, 13)))
    at = jnp.pad(atom_types.astype(I32), (0, NP - N)).reshape(NP, 1)
    bi = jnp.pad(batch_idx.astype(I32), (0, NP - N)).reshape(NP, 1)
    row = jnp.concatenate([edge_index[0].astype(I32),
                           jnp.full((EP - E,), NP - 1, I32)])
    col = jnp.concatenate([edge_index[1].astype(I32),
                           jnp.full((EP - E,), NP - 1, I32)])
    bt = jnp.pad(bond_types.astype(I32), (0, EP - E)).reshape(EP, 1)
    t2 = t.astype(F32).reshape(B, 1)
    zeros_tbl = jnp.zeros((NP, HID), F32)

    ae16 = jnp.pad(p['atom_embed'], ((0, 5), (0, 0)))
    cw16 = jnp.pad(p['coord_W'], ((0, 13), (0, 0)))
    cb = p['coord_b'].reshape(1, HID)
    bond8 = jnp.pad(p['bond_embed'], ((0, 2), (0, 0)))

    def r1(v):
        return v.reshape(1, -1)

    Ls = []
    for lp in p['layers']:
        Ls.append(dict(
            wr=lp['We1'][:HID],
            wc=lp['We1'][HID:2 * HID],
            wea=lp['We1'][2 * HID:2 * HID + 32],
            wd=lp['We1'][2 * HID + 32:].reshape(1, HID),
            be1=r1(lp['be1']), we2=lp['We2'], be2=r1(lp['be2']),
            wc1=lp['Wc1'], bc1=r1(lp['bc1']),
            wc2r=lp['Wc2'].reshape(1, 64), bc2=lp['bc2'].reshape(1, 1),
            wn1h=lp['Wn1'][:HID], wn1m=lp['Wn1'][HID:], bn1=r1(lp['bn1']),
            wn2=lp['Wn2'], bn2=r1(lp['bn2']),
            lng=r1(lp['lng']), lnb=r1(lp['lnb']),
        ))

    wp2 = jnp.pad(p['pred_W2'], ((0, 0), (0, HID - 3)))
    bp2 = jnp.pad(p['pred_b2'], (0, HID - 3)).reshape(1, HID)

    te = _tc_te(t2, p['time_W1'], r1(p['time_b1']), p['time_W2'],
                r1(p['time_b2']))
    h, trow, tcol, xtab = _tc_prologue(x16, at, bi, te, ae16, cw16, cb,
                                       Ls[0]['wr'], Ls[0]['wc'])

    for li in range(NL):
        L = Ls[li]
        Lnxt = Ls[(li + 1) % NL]
        g1, g2, gx1, gx2 = _sc_gather(trow, tcol, xtab, row, col)
        pay, cwu = _tc_edge(g1, g2, gx1, gx2, bt, bond8, L['wea'], L['wd'],
                            L['be1'], L['we2'], L['be2'], L['wc1'], L['bc1'],
                            L['wc2r'], L['bc2'])
        pm = _sc_scatter(pay, col, zeros_tbl)
        px = _sc_scatter(cwu, col, zeros_tbl)
        h, trow, tcol, xtab = _tc_node(h, xtab, pm, px, L['wn1h'], L['wn1m'],
                                       L['bn1'], L['wn2'], L['bn2'], L['lng'],
                                       L['lnb'], Lnxt['wr'], Lnxt['wc'])

    out = _tc_pred(h, p['pred_W1'], r1(p['pred_b1']), wp2, bp2)
    return out[:N, :3]


# speedup vs baseline: 2.7503x; 1.7417x over previous
import dataclasses
import functools

import jax
import jax.numpy as jnp
import numpy as np
from jax import lax
from jax.experimental import pallas as pl
from jax.experimental.pallas import tpu as pltpu
from jax.experimental.pallas import tpu_sc as plsc

F32 = jnp.float32
I32 = jnp.int32

HID = 128
TD = 128
NL = 6
N = 10000
E = 320000
B = 64

NP = 10240
EP = 327680
WIN = 128
NWORK = 32
EPW = EP // NWORK
NWIN = EPW // WIN
ROWS_PER_TILE = NP // 16

BE = 4096
BN = 1024


def _silu(x):
    return x * (1.0 / (1.0 + jnp.exp(-x)))


def _dot(a, b):
    return jnp.dot(a, b, preferred_element_type=F32)


@functools.cache
def _sc_mesh():
    return plsc.VectorSubcoreMesh(core_axis_name="c", subcore_axis_name="s")


def _sc_params():
    cp = pltpu.CompilerParams()
    if "needs_layout_passes" in pltpu.CompilerParams.__dataclass_fields__:
        cp = dataclasses.replace(cp, needs_layout_passes=False)
    return cp


@jax.jit
def _sc_gather(trow, tcol, x3flat, row, col):
    slot_scratch = [
        pltpu.VMEM((WIN,), I32),
        pltpu.VMEM((WIN,), I32),
        pltpu.VMEM((WIN, HID), F32),
        pltpu.VMEM((WIN, HID), F32),
        pltpu.VMEM((WIN, 4), F32),
        pltpu.SemaphoreType.DMA,
        pltpu.SemaphoreType.DMA,
    ]

    @functools.partial(
        pl.kernel,
        out_type=(
            jax.ShapeDtypeStruct((EP, HID), F32),
            jax.ShapeDtypeStruct((EP, HID), F32),
            jax.ShapeDtypeStruct((EP, 4), F32),
        ),
        mesh=_sc_mesh(),
        compiler_params=_sc_params(),
        scratch_types=[pltpu.VMEM((NP * 3,), F32)] + slot_scratch + slot_scratch,
    )
    def k(trow_hbm, tcol_hbm, x3_hbm, row_hbm, col_hbm,
          g1_hbm, g2_hbm, gxd_hbm, xt,
          idx1a, idx2a, buf1a, buf2a, bufxa, gsema, wsema,
          idx1b, idx2b, buf1b, buf2b, bufxb, gsemb, wsemb):
        slotA = (idx1a, idx2a, buf1a, buf2a, bufxa, gsema, wsema)
        slotB = (idx1b, idx2b, buf1b, buf2b, bufxb, gsemb, wsemb)
        wid = lax.axis_index("s") * 2 + lax.axis_index("c")
        base = wid * EPW

        pltpu.sync_copy(x3_hbm, xt)

        def load_and_start(off, S):
            idx1, idx2, buf1, buf2, bufx, gsem, wsem = S
            pltpu.sync_copy(row_hbm.at[pl.ds(off, WIN)], idx1)
            pltpu.sync_copy(col_hbm.at[pl.ds(off, WIN)], idx2)
            pltpu.async_copy(trow_hbm.at[idx1], buf1, gsem)
            pltpu.async_copy(tcol_hbm.at[idx2], buf2, gsem)

        def wait_gathers(S):
            idx1, idx2, buf1, buf2, bufx, gsem, wsem = S
            pltpu.make_async_copy(trow_hbm.at[idx1], buf1, gsem).wait()
            pltpu.make_async_copy(tcol_hbm.at[idx2], buf2, gsem).wait()

        def xcompute(S):
            idx1, idx2, buf1, buf2, bufx, gsem, wsem = S
            for g in range(WIN // 16):
                rows = lax.iota(I32, 16) + (g * 16)
                ra = idx1[pl.ds(g * 16, 16)] * 3
                ca = idx2[pl.ds(g * 16, 16)] * 3
                comps = []
                for kk in range(3):
                    xr = plsc.load_gather(xt, [ra + kk])
                    xc = plsc.load_gather(xt, [ca + kk])
                    d = xr - xc
                    comps.append(d)
                    plsc.store_scatter(bufx, [rows, jnp.full((16,), kk, I32)],
                                       d)
                d2 = (comps[0] * comps[0] + comps[1] * comps[1]
                      + comps[2] * comps[2])
                plsc.store_scatter(bufx, [rows, jnp.full((16,), 3, I32)], d2)

        def write_out(off, S):
            idx1, idx2, buf1, buf2, bufx, gsem, wsem = S
            pltpu.async_copy(buf1, g1_hbm.at[pl.ds(off, WIN)], wsem)
            pltpu.async_copy(buf2, g2_hbm.at[pl.ds(off, WIN)], wsem)
            pltpu.async_copy(bufx, gxd_hbm.at[pl.ds(off, WIN)], wsem)

        def drain_writes(S):
            idx1, idx2, buf1, buf2, bufx, gsem, wsem = S
            pltpu.make_async_copy(buf1, g1_hbm.at[pl.ds(0, WIN)], wsem).wait()
            pltpu.make_async_copy(buf2, g2_hbm.at[pl.ds(0, WIN)], wsem).wait()
            pltpu.make_async_copy(bufx, gxd_hbm.at[pl.ds(0, WIN)], wsem).wait()

        @pl.loop(0, NWIN, step=2)
        def _(wv):
            offA = base + wv * WIN
            offB = offA + WIN

            @pl.when(wv > 0)
            def _():
                drain_writes(slotA)

            load_and_start(offA, slotA)

            @pl.when(wv > 0)
            def _():
                drain_writes(slotB)

            load_and_start(offB, slotB)

            wait_gathers(slotA)
            xcompute(slotA)
            write_out(offA, slotA)

            wait_gathers(slotB)
            xcompute(slotB)
            write_out(offB, slotB)

        drain_writes(slotA)
        drain_writes(slotB)

    return k(trow, tcol, x3flat, row, col)


@jax.jit
def _sc_scatter(payload, col, zeros_tbl):
    slot_scratch = [
        pltpu.VMEM((WIN,), I32),
        pltpu.VMEM((WIN, HID), F32),
        pltpu.SemaphoreType.DMA,
        pltpu.SemaphoreType.DMA,
    ]

    @functools.partial(
        pl.kernel,
        out_type=jax.ShapeDtypeStruct((2, NP, HID), F32),
        mesh=_sc_mesh(),
        scratch_types=[pltpu.VMEM_SHARED((NP, HID), F32)]
        + slot_scratch + slot_scratch,
    )
    def k(p_hbm, col_hbm, z_hbm, out_hbm, acc,
          idxa, rowa, lsema, asema, idxb, rowb, lsemb, asemb):
        slotA = (idxa, rowa, lsema, asema)
        slotB = (idxb, rowb, lsemb, asemb)
        c = lax.axis_index("c")
        s = lax.axis_index("s")
        rbase = s * ROWS_PER_TILE
        pltpu.sync_copy(z_hbm.at[pl.ds(rbase, ROWS_PER_TILE)],
                        acc.at[pl.ds(rbase, ROWS_PER_TILE)])
        plsc.subcore_barrier()
        base = c * (EP // 2) + s * EPW

        def load(off, S):
            idxb_, rowb_, lsem, asem = S
            pltpu.async_copy(col_hbm.at[pl.ds(off, WIN)], idxb_, lsem)
            pltpu.async_copy(p_hbm.at[pl.ds(off, WIN)], rowb_, lsem)

        def wait_load(S):
            idxb_, rowb_, lsem, asem = S
            pltpu.make_async_copy(col_hbm.at[pl.ds(0, WIN)], idxb_,
                                  lsem).wait()
            pltpu.make_async_copy(p_hbm.at[pl.ds(0, WIN)], rowb_,
                                  lsem).wait()

        def start_add(S):
            idxb_, rowb_, lsem, asem = S
            pltpu.async_copy(rowb_, acc.at[idxb_], asem, add=True)

        def drain_add(S):
            idxb_, rowb_, lsem, asem = S
            pltpu.make_async_copy(rowb_, acc.at[idxb_], asem).wait()

        @pl.loop(0, NWIN, step=2)
        def _(wv):
            offA = base + wv * WIN
            offB = offA + WIN

            @pl.when(wv > 0)
            def _():
                drain_add(slotA)

            load(offA, slotA)

            @pl.when(wv > 0)
            def _():
                drain_add(slotB)

            load(offB, slotB)

            wait_load(slotA)
            start_add(slotA)
            wait_load(slotB)
            start_add(slotB)

        drain_add(slotA)
        drain_add(slotB)
        plsc.subcore_barrier()
        pltpu.sync_copy(acc.at[pl.ds(rbase, ROWS_PER_TILE)],
                        out_hbm.at[c, pl.ds(rbase, ROWS_PER_TILE)])

    return k(payload, col, zeros_tbl)


def _te_body(t_ref, w1_ref, b1_ref, w2_ref, b2_ref, te_ref):
    tt = t_ref[...]
    half = TD // 2
    k = lax.broadcasted_iota(I32, (B, half), 1).astype(F32)
    freq = jnp.exp(k * (-np.log(10000.0) / (half - 1)))
    emb = tt * freq
    e1 = jnp.concatenate([jnp.sin(emb), jnp.cos(emb)], axis=1)
    te = _dot(_silu(_dot(e1, w1_ref[...]) + b1_ref[...]), w2_ref[...]) + b2_ref[...]
    te_ref[...] = te


@jax.jit
def _tc_te(t2, w1, b1, w2, b2):
    return pl.pallas_call(
        _te_body,
        out_shape=jax.ShapeDtypeStruct((B, HID), F32),
    )(t2, w1, b1, w2, b2)


def _prologue_body(x_ref, at_ref, bi_ref, te_ref, ae_ref, cw_ref, cb_ref,
                   wr_ref, wc_ref, h_ref, trow_ref, tcol_ref, x3_ref):
    x = x_ref[...]
    at = jnp.clip(at_ref[...], 0, 9)
    oha = (at == lax.broadcasted_iota(I32, (BN, 16), 1)).astype(F32)
    ohb = (bi_ref[...] == lax.broadcasted_iota(I32, (BN, B), 1)).astype(F32)
    h = _dot(oha, ae_ref[...]) + _dot(x, cw_ref[...]) + cb_ref[...] \
        + _dot(ohb, te_ref[...])
    h_ref[...] = h
    trow_ref[...] = _dot(h, wr_ref[...])
    tcol_ref[...] = _dot(h, wc_ref[...])
    x3_ref[...] = x[:, :3]


@jax.jit
def _tc_prologue(x16, at, bi, te, ae16, cw16, cb, wr, wc):
    nb = NP // BN
    full = lambda shape: pl.BlockSpec(shape, lambda i: (0,) * len(shape))
    return pl.pallas_call(
        _prologue_body,
        grid=(nb,),
        in_specs=[
            pl.BlockSpec((BN, 16), lambda i: (i, 0)),
            pl.BlockSpec((BN, 1), lambda i: (i, 0)),
            pl.BlockSpec((BN, 1), lambda i: (i, 0)),
            full((B, HID)),
            full((16, HID)),
            full((16, HID)),
            full((1, HID)),
            full((HID, HID)),
            full((HID, HID)),
        ],
        out_specs=[
            pl.BlockSpec((BN, HID), lambda i: (i, 0)),
            pl.BlockSpec((BN, HID), lambda i: (i, 0)),
            pl.BlockSpec((BN, HID), lambda i: (i, 0)),
            pl.BlockSpec((BN, 3), lambda i: (i, 0)),
        ],
        out_shape=[
            jax.ShapeDtypeStruct((NP, HID), F32),
            jax.ShapeDtypeStruct((NP, HID), F32),
            jax.ShapeDtypeStruct((NP, HID), F32),
            jax.ShapeDtypeStruct((NP, 3), F32),
        ],
    )(x16, at, bi, te, ae16, cw16, cb, wr, wc)


def _edge_body(g1_ref, g2_ref, gxd_ref, bt_ref, bond8_ref, wea_ref,
               wd_ref, be1_ref, we2_ref, be2_ref, wc1_ref, bc1_ref, wc2_ref,
               bc2_ref, p_ref, cwu_ref):
    hsum = g1_ref[...] + g2_ref[...]
    dx3 = gxd_ref[:, :3]
    d2 = gxd_ref[:, 3:4]
    dist = jnp.maximum(jnp.sqrt(d2), 1e-6)
    unit = dx3 / dist
    btc = jnp.clip(bt_ref[...], 0, 4)
    oh = (btc == lax.broadcasted_iota(I32, (BE, 8), 1)).astype(F32)
    tbl8 = _dot(bond8_ref[...], wea_ref[...])
    pre = hsum + _dot(oh, tbl8) + dist * wd_ref[...] + be1_ref[...]
    t1 = _silu(pre)
    m = _silu(_dot(t1, we2_ref[...]) + be2_ref[...])
    c1 = _silu(_dot(m, wc1_ref[...]) + bc1_ref[...])
    cw = jnp.sum(c1 * wc2_ref[...], axis=-1, keepdims=True) + bc2_ref[...]
    p_ref[...] = m
    cwu_ref[...] = jnp.pad(cw * unit, ((0, 0), (0, HID - 3)))


@jax.jit
def _tc_edge(g1, g2, gxd, bt, bond8, wea, wd, be1, we2, be2, wc1, bc1,
             wc2r, bc2):
    nb = EP // BE
    full = lambda shape: pl.BlockSpec(shape, lambda i: (0,) * len(shape))
    return pl.pallas_call(
        _edge_body,
        grid=(nb,),
        in_specs=[
            pl.BlockSpec((BE, HID), lambda i: (i, 0)),
            pl.BlockSpec((BE, HID), lambda i: (i, 0)),
            pl.BlockSpec((BE, 4), lambda i: (i, 0)),
            pl.BlockSpec((BE, 1), lambda i: (i, 0)),
            full((8, 32)),
            full((32, HID)),
            full((1, HID)),
            full((1, HID)),
            full((HID, HID)),
            full((1, HID)),
            full((HID, 64)),
            full((1, 64)),
            full((1, 64)),
            full((1, 1)),
        ],
        out_specs=[
            pl.BlockSpec((BE, HID), lambda i: (i, 0)),
            pl.BlockSpec((BE, HID), lambda i: (i, 0)),
        ],
        out_shape=[
            jax.ShapeDtypeStruct((EP, HID), F32),
            jax.ShapeDtypeStruct((EP, HID), F32),
        ],
    )(g1, g2, gxd, bt, bond8, wea, wd, be1, we2, be2, wc1, bc1, wc2r, bc2)


def _node_body(h_ref, x_ref, pm_ref, px_ref, wn1h_ref, wn1m_ref, bn1_ref,
               wn2_ref, bn2_ref, lng_ref, lnb_ref, wr_ref, wc_ref,
               ho_ref, trow_ref, tcol_ref, x3_ref):
    h = h_ref[...]
    pm = pm_ref[...]
    magg = pm[0] + pm[1]
    xd = px_ref[0, :, :3] + px_ref[1, :, :3]
    xn = x_ref[...] + xd
    n1 = _silu(_dot(h, wn1h_ref[...]) + _dot(magg, wn1m_ref[...]) + bn1_ref[...])
    hn = _dot(n1, wn2_ref[...]) + bn2_ref[...]
    s = h + hn
    mu = jnp.mean(s, axis=-1, keepdims=True)
    var = jnp.mean((s - mu) ** 2, axis=-1, keepdims=True)
    ho = (s - mu) / jnp.sqrt(var + 1e-5) * lng_ref[...] + lnb_ref[...]
    ho_ref[...] = ho
    trow_ref[...] = _dot(ho, wr_ref[...])
    tcol_ref[...] = _dot(ho, wc_ref[...])
    x3_ref[...] = xn


@jax.jit
def _tc_node(h, x3, pm, px, wn1h, wn1m, bn1, wn2, bn2, lng, lnb, wr, wc):
    nb = NP // BN
    full = lambda shape: pl.BlockSpec(shape, lambda i: (0,) * len(shape))
    return pl.pallas_call(
        _node_body,
        grid=(nb,),
        in_specs=[
            pl.BlockSpec((BN, HID), lambda i: (i, 0)),
            pl.BlockSpec((BN, 3), lambda i: (i, 0)),
            pl.BlockSpec((2, BN, HID), lambda i: (0, i, 0)),
            pl.BlockSpec((2, BN, HID), lambda i: (0, i, 0)),
            full((HID, HID)),
            full((HID, HID)),
            full((1, HID)),
            full((HID, HID)),
            full((1, HID)),
            full((1, HID)),
            full((1, HID)),
            full((HID, HID)),
            full((HID, HID)),
        ],
        out_specs=[
            pl.BlockSpec((BN, HID), lambda i: (i, 0)),
            pl.BlockSpec((BN, HID), lambda i: (i, 0)),
            pl.BlockSpec((BN, HID), lambda i: (i, 0)),
            pl.BlockSpec((BN, 3), lambda i: (i, 0)),
        ],
        out_shape=[
            jax.ShapeDtypeStruct((NP, HID), F32),
            jax.ShapeDtypeStruct((NP, HID), F32),
            jax.ShapeDtypeStruct((NP, HID), F32),
            jax.ShapeDtypeStruct((NP, 3), F32),
        ],
    )(h, x3, pm, px, wn1h, wn1m, bn1, wn2, bn2, lng, lnb, wr, wc)


def _pred_body(h_ref, w1_ref, b1_ref, w2_ref, b2_ref, o_ref):
    t1 = _silu(_dot(h_ref[...], w1_ref[...]) + b1_ref[...])
    o_ref[...] = _dot(t1, w2_ref[...]) + b2_ref[...]


@jax.jit
def _tc_pred(h, w1, b1, w2p, b2p):
    nb = NP // BN
    full = lambda shape: pl.BlockSpec(shape, lambda i: (0,) * len(shape))
    return pl.pallas_call(
        _pred_body,
        grid=(nb,),
        in_specs=[
            pl.BlockSpec((BN, HID), lambda i: (i, 0)),
            full((HID, HID)),
            full((1, HID)),
            full((HID, HID)),
            full((1, HID)),
        ],
        out_specs=pl.BlockSpec((BN, HID), lambda i: (i, 0)),
        out_shape=jax.ShapeDtypeStruct((NP, HID), F32),
    )(h, w1, b1, w2p, b2p)


def kernel(x_noisy, t, atom_types, edge_index, bond_types, batch_idx, params):
    p = params

    x16 = jnp.pad(x_noisy.astype(F32), ((0, NP - N), (0, 13)))
    at = jnp.pad(atom_types.astype(I32), (0, NP - N)).reshape(NP, 1)
    bi = jnp.pad(batch_idx.astype(I32), (0, NP - N)).reshape(NP, 1)
    row = jnp.concatenate([edge_index[0].astype(I32),
                           jnp.full((EP - E,), NP - 1, I32)])
    col = jnp.concatenate([edge_index[1].astype(I32),
                           jnp.full((EP - E,), NP - 1, I32)])
    bt = jnp.pad(bond_types.astype(I32), (0, EP - E)).reshape(EP, 1)
    t2 = t.astype(F32).reshape(B, 1)
    zeros_tbl = jnp.zeros((NP, HID), F32)

    ae16 = jnp.pad(p['atom_embed'], ((0, 5), (0, 0)))
    cw16 = jnp.pad(p['coord_W'], ((0, 13), (0, 0)))
    cb = p['coord_b'].reshape(1, HID)
    bond8 = jnp.pad(p['bond_embed'], ((0, 2), (0, 0)))

    def r1(v):
        return v.reshape(1, -1)

    Ls = []
    for lp in p['layers']:
        Ls.append(dict(
            wr=lp['We1'][:HID],
            wc=lp['We1'][HID:2 * HID],
            wea=lp['We1'][2 * HID:2 * HID + 32],
            wd=lp['We1'][2 * HID + 32:].reshape(1, HID),
            be1=r1(lp['be1']), we2=lp['We2'], be2=r1(lp['be2']),
            wc1=lp['Wc1'], bc1=r1(lp['bc1']),
            wc2r=lp['Wc2'].reshape(1, 64), bc2=lp['bc2'].reshape(1, 1),
            wn1h=lp['Wn1'][:HID], wn1m=lp['Wn1'][HID:], bn1=r1(lp['bn1']),
            wn2=lp['Wn2'], bn2=r1(lp['bn2']),
            lng=r1(lp['lng']), lnb=r1(lp['lnb']),
        ))

    wp2 = jnp.pad(p['pred_W2'], ((0, 0), (0, HID - 3)))
    bp2 = jnp.pad(p['pred_b2'], (0, HID - 3)).reshape(1, HID)

    te = _tc_te(t2, p['time_W1'], r1(p['time_b1']), p['time_W2'],
                r1(p['time_b2']))
    h, trow, tcol, x3 = _tc_prologue(x16, at, bi, te, ae16, cw16, cb,
                                     Ls[0]['wr'], Ls[0]['wc'])

    for li in range(NL):
        L = Ls[li]
        Lnxt = Ls[(li + 1) % NL]
        g1, g2, gxd = _sc_gather(trow, tcol, x3.reshape(NP * 3), row, col)
        pay, cwu = _tc_edge(g1, g2, gxd, bt, bond8, L['wea'], L['wd'],
                            L['be1'], L['we2'], L['be2'], L['wc1'], L['bc1'],
                            L['wc2r'], L['bc2'])
        pm = _sc_scatter(pay, col, zeros_tbl)
        px = _sc_scatter(cwu, col, zeros_tbl)
        h, trow, tcol, x3 = _tc_node(h, x3, pm, px, L['wn1h'], L['wn1m'],
                                     L['bn1'], L['wn2'], L['bn2'], L['lng'],
                                     L['lnb'], Lnxt['wr'], Lnxt['wc'])

    out = _tc_pred(h, p['pred_W1'], r1(p['pred_b1']), wp2, bp2)
    return out[:N, :3]


# speedup vs baseline: 3.5350x; 1.2853x over previous
import dataclasses
import functools

import jax
import jax.numpy as jnp
import numpy as np
from jax import lax
from jax.experimental import pallas as pl
from jax.experimental.pallas import tpu as pltpu
from jax.experimental.pallas import tpu_sc as plsc

F32 = jnp.float32
I32 = jnp.int32

HID = 128
TD = 128
NL = 6
N = 10000
E = 320000
B = 64

NP = 10240
EP = 327680
WIN = 128
NWORK = 32
EPW = EP // NWORK
NWIN = EPW // WIN
SWIN = 64
NSWIN = EPW // SWIN
GWIN = 64
EPH = EP // 2
EPWH = EPH // NWORK
NGWIN = EPWH // GWIN
NSWIN2 = EPWH // SWIN
ROWS_PER_TILE = NP // 16

BE = 4096
BN = 1024


def _silu(x):
    return x * (1.0 / (1.0 + jnp.exp(-x)))


def _dot(a, b):
    return jnp.dot(a, b, preferred_element_type=F32)


@functools.cache
def _sc_mesh():
    return plsc.VectorSubcoreMesh(core_axis_name="c", subcore_axis_name="s")


def _sc_params():
    cp = pltpu.CompilerParams()
    if "needs_layout_passes" in pltpu.CompilerParams.__dataclass_fields__:
        cp = dataclasses.replace(cp, needs_layout_passes=False)
    return cp


@functools.partial(jax.jit, static_argnums=(5,))
def _sc_gather(trow, tcol, x3flat, row, col, eoff):
    slot_scratch = [
        pltpu.VMEM((GWIN, HID), F32),
        pltpu.VMEM((GWIN, HID), F32),
        pltpu.VMEM((GWIN, HID), F32),
        pltpu.VMEM((GWIN, 4), F32),
        pltpu.SemaphoreType.DMA,
        pltpu.SemaphoreType.DMA,
    ]

    @functools.partial(
        pl.kernel,
        out_type=(
            jax.ShapeDtypeStruct((EPH, HID), F32),
            jax.ShapeDtypeStruct((EPH, 4), F32),
        ),
        mesh=_sc_mesh(),
        compiler_params=_sc_params(),
        scratch_types=[
            pltpu.VMEM((NP * 3,), F32),
            pltpu.VMEM((EPWH,), I32),
            pltpu.VMEM((EPWH,), I32),
            pltpu.SemaphoreType.DMA,
        ] + slot_scratch + slot_scratch,
    )
    def k(trow_hbm, tcol_hbm, x3_hbm, row_hbm, col_hbm,
          gs_hbm, gxd_hbm, xt, idxr, idxc, psem,
          buf1a, buf2a, obufa, bufxa, gsema, wsema,
          buf1b, buf2b, obufb, bufxb, gsemb, wsemb):
        slotA = (buf1a, buf2a, obufa, bufxa, gsema, wsema)
        slotB = (buf1b, buf2b, obufb, bufxb, gsemb, wsemb)
        wid = lax.axis_index("s") * 2 + lax.axis_index("c")
        base = wid * EPWH

        cpx = pltpu.async_copy(x3_hbm, xt, psem)
        cpr = pltpu.async_copy(row_hbm.at[pl.ds(eoff + base, EPWH)], idxr,
                               psem)
        cpc = pltpu.async_copy(col_hbm.at[pl.ds(eoff + base, EPWH)], idxc,
                               psem)
        cpx.wait()
        cpr.wait()
        cpc.wait()

        def start(w, S):
            buf1, buf2, obuf, bufx, gsem, wsem = S
            loc = w * GWIN
            pltpu.async_copy(trow_hbm.at[idxr.at[pl.ds(loc, GWIN)]], buf1,
                             gsem)
            pltpu.async_copy(tcol_hbm.at[idxc.at[pl.ds(loc, GWIN)]], buf2,
                             gsem)

        def wait_gathers(w, S):
            buf1, buf2, obuf, bufx, gsem, wsem = S
            loc = w * GWIN
            pltpu.make_async_copy(trow_hbm.at[idxr.at[pl.ds(loc, GWIN)]],
                                  buf1, gsem).wait()
            pltpu.make_async_copy(tcol_hbm.at[idxc.at[pl.ds(loc, GWIN)]],
                                  buf2, gsem).wait()

        def compute(w, S):
            buf1, buf2, obuf, bufx, gsem, wsem = S
            loc = w * GWIN

            @pl.loop(0, GWIN)
            def _(r):
                for c8 in range(HID // 16):
                    slc = (r, pl.ds(c8 * 16, 16))
                    obuf[slc] = buf1[slc] + buf2[slc]

            for g in range(GWIN // 16):
                rows = lax.iota(I32, 16) + (g * 16)
                ra = idxr[pl.ds(loc + g * 16, 16)] * 3
                ca = idxc[pl.ds(loc + g * 16, 16)] * 3
                comps = []
                for kk in range(3):
                    xr = plsc.load_gather(xt, [ra + kk])
                    xc = plsc.load_gather(xt, [ca + kk])
                    d = xr - xc
                    comps.append(d)
                    plsc.store_scatter(bufx, [rows, jnp.full((16,), kk, I32)],
                                       d)
                d2 = (comps[0] * comps[0] + comps[1] * comps[1]
                      + comps[2] * comps[2])
                plsc.store_scatter(bufx, [rows, jnp.full((16,), 3, I32)], d2)

        def write_out(w, S):
            buf1, buf2, obuf, bufx, gsem, wsem = S
            off = base + w * GWIN
            pltpu.async_copy(obuf, gs_hbm.at[pl.ds(off, GWIN)], wsem)
            pltpu.async_copy(bufx, gxd_hbm.at[pl.ds(off, GWIN)], wsem)

        def drain_writes(S):
            buf1, buf2, obuf, bufx, gsem, wsem = S
            pltpu.make_async_copy(obuf, gs_hbm.at[pl.ds(0, GWIN)],
                                  wsem).wait()
            pltpu.make_async_copy(bufx, gxd_hbm.at[pl.ds(0, GWIN)],
                                  wsem).wait()

        start(0, slotA)
        start(1, slotB)

        @pl.loop(0, NGWIN, step=2)
        def _(w):
            wait_gathers(w, slotA)

            @pl.when(w > 0)
            def _():
                drain_writes(slotA)

            compute(w, slotA)

            @pl.when(w + 2 < NGWIN)
            def _():
                start(w + 2, slotA)

            write_out(w, slotA)

            wait_gathers(w + 1, slotB)

            @pl.when(w > 0)
            def _():
                drain_writes(slotB)

            compute(w + 1, slotB)

            @pl.when(w + 3 < NGWIN)
            def _():
                start(w + 3, slotB)

            write_out(w + 1, slotB)

        drain_writes(slotA)
        drain_writes(slotB)

    return k(trow, tcol, x3flat, row, col)


@functools.partial(jax.jit, static_argnums=(5,))
def _sc_scatter(payload, cwu4, col, init_m, init_x, eoff):
    slot_scratch = [
        pltpu.VMEM((SWIN,), I32),
        pltpu.VMEM((SWIN, HID), F32),
        pltpu.VMEM((SWIN, 4), F32),
        pltpu.VMEM((SWIN,), F32),
        pltpu.VMEM((SWIN,), F32),
        pltpu.VMEM((SWIN,), F32),
        pltpu.VMEM((SWIN,), I32),
        pltpu.VMEM((SWIN,), I32),
        pltpu.VMEM((SWIN,), I32),
        pltpu.SemaphoreType.DMA,
        pltpu.SemaphoreType.DMA,
    ]

    @functools.partial(
        pl.kernel,
        out_type=(
            jax.ShapeDtypeStruct((2, NP, HID), F32),
            jax.ShapeDtypeStruct((2, NP * 4), F32),
        ),
        mesh=_sc_mesh(),
        compiler_params=_sc_params(),
        scratch_types=[
            pltpu.VMEM_SHARED((NP, HID), F32),
            pltpu.VMEM_SHARED((NP * 4,), F32),
        ] + slot_scratch + slot_scratch,
    )
    def k(p_hbm, c_hbm, col_hbm, z_hbm, zx_hbm, out_hbm, outx_hbm, acc, accx,
          *slots):
        slotA = slots[:11]
        slotB = slots[11:]
        c = lax.axis_index("c")
        s = lax.axis_index("s")
        rbase = s * ROWS_PER_TILE
        xb = s * (NP * 4 // 16)
        pltpu.sync_copy(z_hbm.at[c, pl.ds(rbase, ROWS_PER_TILE)],
                        acc.at[pl.ds(rbase, ROWS_PER_TILE)])
        pltpu.sync_copy(zx_hbm.at[c, pl.ds(xb, NP * 4 // 16)],
                        accx.at[pl.ds(xb, NP * 4 // 16)])
        plsc.subcore_barrier()
        base = c * (EPH // 2) + s * EPWH

        def load(off, S):
            idxb, rowb, cb, vx, vy, vz, ix, iy, iz, lsem, asem = S
            pltpu.async_copy(col_hbm.at[pl.ds(eoff + off, SWIN)], idxb, lsem)
            pltpu.async_copy(p_hbm.at[pl.ds(off, SWIN)], rowb, lsem)
            pltpu.async_copy(c_hbm.at[pl.ds(off, SWIN)], cb, lsem)

        def wait_load(S):
            idxb, rowb, cb, vx, vy, vz, ix, iy, iz, lsem, asem = S
            pltpu.make_async_copy(col_hbm.at[pl.ds(0, SWIN)], idxb,
                                  lsem).wait()
            pltpu.make_async_copy(p_hbm.at[pl.ds(0, SWIN)], rowb,
                                  lsem).wait()
            pltpu.make_async_copy(c_hbm.at[pl.ds(0, SWIN)], cb, lsem).wait()

        def start_add(S):
            idxb, rowb, cb, vx, vy, vz, ix, iy, iz, lsem, asem = S
            pltpu.async_copy(rowb, acc.at[idxb], asem, add=True)
            for g in range(SWIN // 16):
                rows = lax.iota(I32, 16) + (g * 16)
                ci4 = idxb[pl.ds(g * 16, 16)] * 4
                for kk, (vb, ib) in enumerate(((vx, ix), (vy, iy), (vz, iz))):
                    v = plsc.load_gather(cb, [rows, jnp.full((16,), kk, I32)])
                    vb[pl.ds(g * 16, 16)] = v
                    ib[pl.ds(g * 16, 16)] = ci4 + kk
            pltpu.async_copy(vx, accx.at[ix], asem, add=True)
            pltpu.async_copy(vy, accx.at[iy], asem, add=True)
            pltpu.async_copy(vz, accx.at[iz], asem, add=True)

        def drain_add(S):
            idxb, rowb, cb, vx, vy, vz, ix, iy, iz, lsem, asem = S
            pltpu.make_async_copy(rowb, acc.at[idxb], asem).wait()
            pltpu.make_async_copy(vx, accx.at[ix], asem).wait()
            pltpu.make_async_copy(vy, accx.at[iy], asem).wait()
            pltpu.make_async_copy(vz, accx.at[iz], asem).wait()

        @pl.loop(0, NSWIN2, step=2)
        def _(wv):
            offA = base + wv * SWIN
            offB = offA + SWIN

            @pl.when(wv > 0)
            def _():
                drain_add(slotA)

            load(offA, slotA)

            @pl.when(wv > 0)
            def _():
                drain_add(slotB)

            load(offB, slotB)

            wait_load(slotA)
            start_add(slotA)
            wait_load(slotB)
            start_add(slotB)

        drain_add(slotA)
        drain_add(slotB)
        plsc.subcore_barrier()
        pltpu.sync_copy(acc.at[pl.ds(rbase, ROWS_PER_TILE)],
                        out_hbm.at[c, pl.ds(rbase, ROWS_PER_TILE)])
        pltpu.sync_copy(accx.at[pl.ds(xb, NP * 4 // 16)],
                        outx_hbm.at[c, pl.ds(xb, NP * 4 // 16)])

    return k(payload, cwu4, col, init_m, init_x)


def _te_body(t_ref, w1_ref, b1_ref, w2_ref, b2_ref, te_ref):
    tt = t_ref[...]
    half = TD // 2
    k = lax.broadcasted_iota(I32, (B, half), 1).astype(F32)
    freq = jnp.exp(k * (-np.log(10000.0) / (half - 1)))
    emb = tt * freq
    e1 = jnp.concatenate([jnp.sin(emb), jnp.cos(emb)], axis=1)
    te = _dot(_silu(_dot(e1, w1_ref[...]) + b1_ref[...]), w2_ref[...]) + b2_ref[...]
    te_ref[...] = te


@jax.jit
def _tc_te(t2, w1, b1, w2, b2):
    return pl.pallas_call(
        _te_body,
        out_shape=jax.ShapeDtypeStruct((B, HID), F32),
    )(t2, w1, b1, w2, b2)


def _prologue_body(x_ref, at_ref, bi_ref, te_ref, ae_ref, cw_ref, cb_ref,
                   wr_ref, wc_ref, h_ref, trow_ref, tcol_ref, x3_ref):
    x = x_ref[...]
    at = jnp.clip(at_ref[...], 0, 9)
    oha = (at == lax.broadcasted_iota(I32, (BN, 16), 1)).astype(F32)
    ohb = (bi_ref[...] == lax.broadcasted_iota(I32, (BN, B), 1)).astype(F32)
    h = _dot(oha, ae_ref[...]) + _dot(x, cw_ref[...]) + cb_ref[...] \
        + _dot(ohb, te_ref[...])
    h_ref[...] = h
    trow_ref[...] = _dot(h, wr_ref[...])
    tcol_ref[...] = _dot(h, wc_ref[...])
    x3_ref[...] = x[:, :3]


@jax.jit
def _tc_prologue(x16, at, bi, te, ae16, cw16, cb, wr, wc):
    nb = NP // BN
    full = lambda shape: pl.BlockSpec(shape, lambda i: (0,) * len(shape))
    return pl.pallas_call(
        _prologue_body,
        grid=(nb,),
        in_specs=[
            pl.BlockSpec((BN, 16), lambda i: (i, 0)),
            pl.BlockSpec((BN, 1), lambda i: (i, 0)),
            pl.BlockSpec((BN, 1), lambda i: (i, 0)),
            full((B, HID)),
            full((16, HID)),
            full((16, HID)),
            full((1, HID)),
            full((HID, HID)),
            full((HID, HID)),
        ],
        out_specs=[
            pl.BlockSpec((BN, HID), lambda i: (i, 0)),
            pl.BlockSpec((BN, HID), lambda i: (i, 0)),
            pl.BlockSpec((BN, HID), lambda i: (i, 0)),
            pl.BlockSpec((BN, 3), lambda i: (i, 0)),
        ],
        out_shape=[
            jax.ShapeDtypeStruct((NP, HID), F32),
            jax.ShapeDtypeStruct((NP, HID), F32),
            jax.ShapeDtypeStruct((NP, HID), F32),
            jax.ShapeDtypeStruct((NP, 3), F32),
        ],
    )(x16, at, bi, te, ae16, cw16, cb, wr, wc)


def _edge_body(gs_ref, gxd_ref, bt_ref, bond8_ref, wea_ref,
               wd_ref, be1_ref, we2_ref, be2_ref, wc1_ref, bc1_ref, wc2_ref,
               bc2_ref, p_ref, cwu_ref):
    hsum = gs_ref[...]
    dx3 = gxd_ref[:, :3]
    d2 = gxd_ref[:, 3:4]
    dist = jnp.maximum(jnp.sqrt(d2), 1e-6)
    unit = dx3 / dist
    btc = jnp.clip(bt_ref[...], 0, 4)
    oh = (btc == lax.broadcasted_iota(I32, (BE, 8), 1)).astype(F32)
    tbl8 = _dot(bond8_ref[...], wea_ref[...])
    pre = hsum + _dot(oh, tbl8) + dist * wd_ref[...] + be1_ref[...]
    t1 = _silu(pre)
    m = _silu(_dot(t1, we2_ref[...]) + be2_ref[...])
    c1 = _silu(_dot(m, wc1_ref[...]) + bc1_ref[...])
    cw = jnp.sum(c1 * wc2_ref[...], axis=-1, keepdims=True) + bc2_ref[...]
    p_ref[...] = m
    cwu_ref[...] = jnp.pad(cw * unit, ((0, 0), (0, 1)))


@functools.partial(jax.jit, static_argnums=(13,))
def _tc_edge(gs, gxd, bt, bond8, wea, wd, be1, we2, be2, wc1, bc1,
             wc2r, bc2, boff):
    nb = EPH // BE
    full = lambda shape: pl.BlockSpec(shape, lambda i: (0,) * len(shape))
    return pl.pallas_call(
        _edge_body,
        grid=(nb,),
        in_specs=[
            pl.BlockSpec((BE, HID), lambda i: (i, 0)),
            pl.BlockSpec((BE, 4), lambda i: (i, 0)),
            pl.BlockSpec((BE, 1), lambda i: (i + boff, 0)),
            full((8, 32)),
            full((32, HID)),
            full((1, HID)),
            full((1, HID)),
            full((HID, HID)),
            full((1, HID)),
            full((HID, 64)),
            full((1, 64)),
            full((1, 64)),
            full((1, 1)),
        ],
        out_specs=[
            pl.BlockSpec((BE, HID), lambda i: (i, 0)),
            pl.BlockSpec((BE, 4), lambda i: (i, 0)),
        ],
        out_shape=[
            jax.ShapeDtypeStruct((EPH, HID), F32),
            jax.ShapeDtypeStruct((EPH, 4), F32),
        ],
    )(gs, gxd, bt, bond8, wea, wd, be1, we2, be2, wc1, bc1, wc2r, bc2)


def _node_body(h_ref, x_ref, pm_ref, px_ref, wn1h_ref, wn1m_ref, bn1_ref,
               wn2_ref, bn2_ref, lng_ref, lnb_ref, wr_ref, wc_ref,
               ho_ref, trow_ref, tcol_ref, x3_ref):
    h = h_ref[...]
    pm = pm_ref[...]
    magg = pm[0] + pm[1]
    xd = px_ref[0, :, :3] + px_ref[1, :, :3]
    xn = x_ref[...] + xd
    n1 = _silu(_dot(h, wn1h_ref[...]) + _dot(magg, wn1m_ref[...]) + bn1_ref[...])
    hn = _dot(n1, wn2_ref[...]) + bn2_ref[...]
    s = h + hn
    mu = jnp.mean(s, axis=-1, keepdims=True)
    var = jnp.mean((s - mu) ** 2, axis=-1, keepdims=True)
    ho = (s - mu) / jnp.sqrt(var + 1e-5) * lng_ref[...] + lnb_ref[...]
    ho_ref[...] = ho
    trow_ref[...] = _dot(ho, wr_ref[...])
    tcol_ref[...] = _dot(ho, wc_ref[...])
    x3_ref[...] = xn


@jax.jit
def _tc_node(h, x3, pm, px, wn1h, wn1m, bn1, wn2, bn2, lng, lnb, wr, wc):
    nb = NP // BN
    full = lambda shape: pl.BlockSpec(shape, lambda i: (0,) * len(shape))
    return pl.pallas_call(
        _node_body,
        grid=(nb,),
        in_specs=[
            pl.BlockSpec((BN, HID), lambda i: (i, 0)),
            pl.BlockSpec((BN, 3), lambda i: (i, 0)),
            pl.BlockSpec((2, BN, HID), lambda i: (0, i, 0)),
            pl.BlockSpec((2, BN, 4), lambda i: (0, i, 0)),
            full((HID, HID)),
            full((HID, HID)),
            full((1, HID)),
            full((HID, HID)),
            full((1, HID)),
            full((1, HID)),
            full((1, HID)),
            full((HID, HID)),
            full((HID, HID)),
        ],
        out_specs=[
            pl.BlockSpec((BN, HID), lambda i: (i, 0)),
            pl.BlockSpec((BN, HID), lambda i: (i, 0)),
            pl.BlockSpec((BN, HID), lambda i: (i, 0)),
            pl.BlockSpec((BN, 3), lambda i: (i, 0)),
        ],
        out_shape=[
            jax.ShapeDtypeStruct((NP, HID), F32),
            jax.ShapeDtypeStruct((NP, HID), F32),
            jax.ShapeDtypeStruct((NP, HID), F32),
            jax.ShapeDtypeStruct((NP, 3), F32),
        ],
    )(h, x3, pm, px, wn1h, wn1m, bn1, wn2, bn2, lng, lnb, wr, wc)


def _pred_body(h_ref, w1_ref, b1_ref, w2_ref, b2_ref, o_ref):
    t1 = _silu(_dot(h_ref[...], w1_ref[...]) + b1_ref[...])
    o_ref[...] = _dot(t1, w2_ref[...]) + b2_ref[...]


@jax.jit
def _tc_pred(h, w1, b1, w2p, b2p):
    nb = NP // BN
    full = lambda shape: pl.BlockSpec(shape, lambda i: (0,) * len(shape))
    return pl.pallas_call(
        _pred_body,
        grid=(nb,),
        in_specs=[
            pl.BlockSpec((BN, HID), lambda i: (i, 0)),
            full((HID, HID)),
            full((1, HID)),
            full((HID, HID)),
            full((1, HID)),
        ],
        out_specs=pl.BlockSpec((BN, HID), lambda i: (i, 0)),
        out_shape=jax.ShapeDtypeStruct((NP, HID), F32),
    )(h, w1, b1, w2p, b2p)


def kernel(x_noisy, t, atom_types, edge_index, bond_types, batch_idx, params):
    p = params

    x16 = jnp.pad(x_noisy.astype(F32), ((0, NP - N), (0, 13)))
    at = jnp.pad(atom_types.astype(I32), (0, NP - N)).reshape(NP, 1)
    bi = jnp.pad(batch_idx.astype(I32), (0, NP - N)).reshape(NP, 1)
    row = jnp.concatenate([edge_index[0].astype(I32),
                           jnp.full((EP - E,), NP - 1, I32)])
    col = jnp.concatenate([edge_index[1].astype(I32),
                           jnp.full((EP - E,), NP - 1, I32)])
    bt = jnp.pad(bond_types.astype(I32), (0, EP - E)).reshape(EP, 1)
    t2 = t.astype(F32).reshape(B, 1)
    zeros_tbl = jnp.zeros((2, NP, HID), F32)
    zerosx = jnp.zeros((2, NP * 4), F32)

    ae16 = jnp.pad(p['atom_embed'], ((0, 5), (0, 0)))
    cw16 = jnp.pad(p['coord_W'], ((0, 13), (0, 0)))
    cb = p['coord_b'].reshape(1, HID)
    bond8 = jnp.pad(p['bond_embed'], ((0, 2), (0, 0)))

    def r1(v):
        return v.reshape(1, -1)

    Ls = []
    for lp in p['layers']:
        Ls.append(dict(
            wr=lp['We1'][:HID],
            wc=lp['We1'][HID:2 * HID],
            wea=lp['We1'][2 * HID:2 * HID + 32],
            wd=lp['We1'][2 * HID + 32:].reshape(1, HID),
            be1=r1(lp['be1']), we2=lp['We2'], be2=r1(lp['be2']),
            wc1=lp['Wc1'], bc1=r1(lp['bc1']),
            wc2r=lp['Wc2'].reshape(1, 64), bc2=lp['bc2'].reshape(1, 1),
            wn1h=lp['Wn1'][:HID], wn1m=lp['Wn1'][HID:], bn1=r1(lp['bn1']),
            wn2=lp['Wn2'], bn2=r1(lp['bn2']),
            lng=r1(lp['lng']), lnb=r1(lp['lnb']),
        ))

    wp2 = jnp.pad(p['pred_W2'], ((0, 0), (0, HID - 3)))
    bp2 = jnp.pad(p['pred_b2'], (0, HID - 3)).reshape(1, HID)

    te = _tc_te(t2, p['time_W1'], r1(p['time_b1']), p['time_W2'],
                r1(p['time_b2']))
    h, trow, tcol, x3 = _tc_prologue(x16, at, bi, te, ae16, cw16, cb,
                                     Ls[0]['wr'], Ls[0]['wc'])

    for li in range(NL):
        L = Ls[li]
        Lnxt = Ls[(li + 1) % NL]
        x3f = x3.reshape(NP * 3)
        gs1, gxd1 = _sc_gather(trow, tcol, x3f, row, col, 0)
        gs2, gxd2 = _sc_gather(trow, tcol, x3f, row, col, EPH)
        ew = (bt, bond8, L['wea'], L['wd'], L['be1'], L['we2'], L['be2'],
              L['wc1'], L['bc1'], L['wc2r'], L['bc2'])
        pay1, cwu1 = _tc_edge(gs1, gxd1, *ew, 0)
        pay2, cwu2 = _tc_edge(gs2, gxd2, *ew, EPH // BE)
        pm1, pxf1 = _sc_scatter(pay1, cwu1, col, zeros_tbl, zerosx, 0)
        pm, pxf = _sc_scatter(pay2, cwu2, col, pm1, pxf1, EPH)
        px = pxf.reshape(2, NP, 4)
        h, trow, tcol, x3 = _tc_node(h, x3, pm, px, L['wn1h'], L['wn1m'],
                                     L['bn1'], L['wn2'], L['bn2'], L['lng'],
                                     L['lnb'], Lnxt['wr'], Lnxt['wc'])

    out = _tc_pred(h, p['pred_W1'], r1(p['pred_b1']), wp2, bp2)
    return out[:N, :3]
